# baseline (speedup 1.0000x reference)
import functools

import jax
import jax.numpy as jnp
from jax import lax
from jax.experimental import pallas as pl
from jax.experimental.pallas import tpu as pltpu
from jax.experimental.pallas import tpu_sc as plsc

N = 10000
D = 128
E = 320000

NC, NS = 2, 16
NW = NC * NS
NPAD = 10240
RPT = NPAD // NS
K = 50
NCHUNK = 200
EPT = E // NW
NBUF = 4
HALF = NBUF // 2
NROUND = NCHUNK // NBUF
DBUF = 8
DROUND = NCHUNK // DBUF

_mesh = plsc.VectorSubcoreMesh(core_axis_name="c", subcore_axis_name="s")


@functools.partial(
    pl.kernel,
    out_type=jax.ShapeDtypeStruct((NC, NPAD), jnp.float32),
    mesh=_mesh,
    scratch_types=[
        pltpu.VMEM((NCHUNK, K), jnp.int32),
        pltpu.VMEM((64,), jnp.float32),
        pltpu.VMEM((RPT,), jnp.float32),
        pltpu.VMEM_SHARED((NPAD,), jnp.float32),
        pltpu.SemaphoreType.DMA((DBUF,)),
    ],
)
def _degree_kernel(src2d_hbm, out_hbm, idx_v, ones_v, zero_v, hist_sh, sems):
    c = lax.axis_index("c")
    s = lax.axis_index("s")
    wid = s * NC + c

    for j in range(RPT // 16):
        zero_v[pl.ds(16 * j, 16)] = jnp.zeros((16,), jnp.float32)
    for j in range(4):
        ones_v[pl.ds(16 * j, 16)] = jnp.ones((16,), jnp.float32)
    pltpu.sync_copy(src2d_hbm.at[pl.ds(wid * NCHUNK, NCHUNK)], idx_v)
    pltpu.sync_copy(zero_v, hist_sh.at[pl.ds(s * RPT, RPT)])
    plsc.subcore_barrier()

    def body(g, carry):
        i0 = g * DBUF
        for b in range(DBUF):
            @pl.when(g > 0)
            def _():
                pltpu.make_async_copy(
                    ones_v.at[pl.ds(0, K)],
                    hist_sh.at[idx_v.at[i0 - DBUF + b]], sems.at[b]).wait()
            pltpu.async_copy(ones_v.at[pl.ds(0, K)],
                             hist_sh.at[idx_v.at[i0 + b]], sems.at[b],
                             add=True)
        return carry

    lax.fori_loop(0, DROUND, body, 0)
    for b in range(DBUF):
        pltpu.make_async_copy(
            ones_v.at[pl.ds(0, K)],
            hist_sh.at[idx_v.at[NCHUNK - DBUF + b]], sems.at[b]).wait()
    plsc.subcore_barrier()

    pltpu.sync_copy(hist_sh.at[pl.ds(s * RPT, RPT)],
                    out_hbm.at[c, pl.ds(s * RPT, RPT)])


BR = 2000


def _scale_body(da_ref, db_ref, x_ref, xs_ref, r_ref):
    deg = jnp.maximum(da_ref[...] + db_ref[...], 1.0)
    r = lax.rsqrt(deg)
    r_ref[...] = r
    xs_ref[...] = x_ref[...] * r


def _scale_call(da, db, x):
    return pl.pallas_call(
        _scale_body,
        grid=(N // BR,),
        in_specs=[
            pl.BlockSpec((BR, 1), lambda i: (i, 0)),
            pl.BlockSpec((BR, 1), lambda i: (i, 0)),
            pl.BlockSpec((BR, D), lambda i: (i, 0)),
        ],
        out_specs=[
            pl.BlockSpec((BR, D), lambda i: (i, 0)),
            pl.BlockSpec((BR, 1), lambda i: (i, 0)),
        ],
        out_shape=[
            jax.ShapeDtypeStruct((N, D), jnp.float32),
            jax.ShapeDtypeStruct((N, 1), jnp.float32),
        ],
    )(da, db, x)


@functools.partial(
    pl.kernel,
    out_type=jax.ShapeDtypeStruct((NC, NPAD, D), jnp.float32),
    mesh=_mesh,
    scratch_types=[
        pltpu.VMEM((NBUF, K), jnp.int32),
        pltpu.VMEM((NBUF, K), jnp.int32),
        pltpu.VMEM((NBUF, K, D), jnp.float32),
        pltpu.VMEM((32, D), jnp.float32),
        pltpu.VMEM_SHARED((NPAD, D), jnp.float32),
        pltpu.SemaphoreType.DMA((NBUF,)),
        pltpu.SemaphoreType.DMA((NBUF,)),
        pltpu.SemaphoreType.DMA((NBUF,)),
    ],
)
def _agg_kernel(xs_hbm, src2d_hbm, dst2d_hbm, out_hbm,
                sidx_v, didx_v, rows_v, zero_v, agg_sh, gsem, ssem, isem):
    c = lax.axis_index("c")
    s = lax.axis_index("s")
    wid = s * NC + c
    r0 = s * RPT
    row0 = wid * NCHUNK

    for jr in range(32):
        for jc in range(D // 16):
            zero_v[jr, pl.ds(16 * jc, 16)] = jnp.zeros((16,), jnp.float32)
    for j in range(RPT // 32):
        pltpu.sync_copy(zero_v,
                        agg_sh.at[pl.ds(pl.multiple_of(r0 + 32 * j, 32), 32)])
    plsc.subcore_barrier()

    def idx_load(i, b):
        pltpu.async_copy(src2d_hbm.at[row0 + i], sidx_v.at[b], isem.at[b])
        pltpu.async_copy(dst2d_hbm.at[row0 + i], didx_v.at[b], isem.at[b])

    def idx_wait(i, b):
        pltpu.make_async_copy(src2d_hbm.at[row0 + i], sidx_v.at[b],
                              isem.at[b]).wait()
        pltpu.make_async_copy(dst2d_hbm.at[row0 + i], didx_v.at[b],
                              isem.at[b]).wait()

    def gather(b):
        pltpu.async_copy(xs_hbm.at[sidx_v.at[b]], rows_v.at[b], gsem.at[b])

    def gather_wait(b):
        pltpu.make_async_copy(xs_hbm.at[sidx_v.at[b]], rows_v.at[b],
                              gsem.at[b]).wait()

    def scatter(b):
        pltpu.async_copy(rows_v.at[b], agg_sh.at[didx_v.at[b]], ssem.at[b],
                         add=True)

    def scatter_wait(b):
        pltpu.make_async_copy(rows_v.at[b], agg_sh.at[didx_v.at[b]],
                              ssem.at[b]).wait()

    A = list(range(HALF))
    B = list(range(HALF, NBUF))

    for b in A:
        idx_load(b, b)
    for b in A:
        idx_wait(b, b)
        gather(b)
    for b in B:
        idx_load(b, b)

    def body(g, carry):
        i0 = g * NBUF
        for b in A:
            gather_wait(b)
            scatter(b)
        for b in B:
            idx_wait(i0 + b, b)
            gather(b)
        for b in A:
            scatter_wait(b)
            @pl.when(g < NROUND - 1)
            def _():
                idx_load(i0 + NBUF + b, b)
        for b in B:
            gather_wait(b)
            scatter(b)
        for b in A:
            @pl.when(g < NROUND - 1)
            def _():
                idx_wait(i0 + NBUF + b, b)
                gather(b)
        for b in B:
            scatter_wait(b)
            @pl.when(g < NROUND - 1)
            def _():
                idx_load(i0 + NBUF + b, b)
        return carry

    lax.fori_loop(0, NROUND, body, 0)
    plsc.subcore_barrier()

    pltpu.sync_copy(agg_sh.at[pl.ds(r0, RPT)], out_hbm.at[c, pl.ds(r0, RPT)])


def _ffn_body(p0_ref, p1_ref, r_ref, w1_ref, b1_ref, w2_ref, b2_ref, out_ref):
    a = (p0_ref[0] + p1_ref[0]) * r_ref[...]
    h = jnp.maximum(jnp.dot(a, w1_ref[...],
                            preferred_element_type=jnp.float32) + b1_ref[...],
                    0.0)
    out_ref[...] = jnp.dot(h, w2_ref[...],
                           preferred_element_type=jnp.float32) + b2_ref[...]


def _ffn_call(agg_p, r, w1, b1, w2, b2):
    full = lambda i: (0, 0)
    return pl.pallas_call(
        _ffn_body,
        grid=(N // BR,),
        in_specs=[
            pl.BlockSpec((1, BR, D), lambda i: (0, i, 0)),
            pl.BlockSpec((1, BR, D), lambda i: (1, i, 0)),
            pl.BlockSpec((BR, 1), lambda i: (i, 0)),
            pl.BlockSpec((D, D), full),
            pl.BlockSpec((1, D), full),
            pl.BlockSpec((D, D), full),
            pl.BlockSpec((1, D), full),
        ],
        out_specs=pl.BlockSpec((BR, D), lambda i: (i, 0)),
        out_shape=jax.ShapeDtypeStruct((N, D), jnp.float32),
    )(agg_p, agg_p, r, w1, b1, w2, b2)


def kernel(x, edge_index, W1, b1, W2, b2):
    src = edge_index[0]
    dst = edge_index[1]

    src_deg2d = src.reshape(NW * NCHUNK, K)
    dst_agg2d = dst.reshape(NW * NCHUNK, K)

    deg_p = _degree_kernel(src_deg2d)
    da = deg_p[0].reshape(NPAD, 1)
    db = deg_p[1].reshape(NPAD, 1)
    xs, r = _scale_call(da, db, x)

    agg_p = _agg_kernel(xs, src_deg2d, dst_agg2d)

    return _ffn_call(agg_p, r,
                     W1, b1.reshape(1, D), W2, b2.reshape(1, D))


# speedup vs baseline: 1.0662x; 1.0662x over previous
import functools

import jax
import jax.numpy as jnp
from jax import lax
from jax.experimental import pallas as pl
from jax.experimental.pallas import tpu as pltpu
from jax.experimental.pallas import tpu_sc as plsc

N = 10000
D = 128
E = 320000

NC, NS = 2, 16
NW = NC * NS
NPAD = 10240
RPT = NPAD // NS
K = 64
NCHUNK = 160
EPT_P = NCHUNK * K
E_P = NW * EPT_P
PAD_E = E_P - E
NBUF = 4
HALF = NBUF // 2
NROUND = NCHUNK // NBUF
DBUF = 8
DROUND = NCHUNK // DBUF

_mesh = plsc.VectorSubcoreMesh(core_axis_name="c", subcore_axis_name="s")


@functools.partial(
    pl.kernel,
    out_type=jax.ShapeDtypeStruct((NC, NPAD), jnp.float32),
    mesh=_mesh,
    scratch_types=[
        pltpu.VMEM((NCHUNK, K), jnp.int32),
        pltpu.VMEM((K,), jnp.float32),
        pltpu.VMEM((RPT,), jnp.float32),
        pltpu.VMEM_SHARED((NPAD,), jnp.float32),
        pltpu.SemaphoreType.DMA((DBUF,)),
    ],
)
def _degree_kernel(src2d_hbm, out_hbm, idx_v, ones_v, zero_v, hist_sh, sems):
    c = lax.axis_index("c")
    s = lax.axis_index("s")
    wid = s * NC + c

    for j in range(RPT // 16):
        zero_v[pl.ds(16 * j, 16)] = jnp.zeros((16,), jnp.float32)
    for j in range(K // 16):
        ones_v[pl.ds(16 * j, 16)] = jnp.ones((16,), jnp.float32)
    pltpu.sync_copy(src2d_hbm.at[pl.ds(wid * NCHUNK, NCHUNK)], idx_v)
    pltpu.sync_copy(zero_v, hist_sh.at[pl.ds(s * RPT, RPT)])
    plsc.subcore_barrier()

    def body(g, carry):
        i0 = g * DBUF
        for b in range(DBUF):
            @pl.when(g > 0)
            def _():
                pltpu.make_async_copy(
                    ones_v, hist_sh.at[idx_v.at[i0 - DBUF + b]],
                    sems.at[b]).wait()
            pltpu.async_copy(ones_v, hist_sh.at[idx_v.at[i0 + b]],
                             sems.at[b], add=True)
        return carry

    lax.fori_loop(0, DROUND, body, 0)
    for b in range(DBUF):
        pltpu.make_async_copy(
            ones_v, hist_sh.at[idx_v.at[NCHUNK - DBUF + b]],
            sems.at[b]).wait()
    plsc.subcore_barrier()

    pltpu.sync_copy(hist_sh.at[pl.ds(s * RPT, RPT)],
                    out_hbm.at[c, pl.ds(s * RPT, RPT)])


BR = 2000


def _scale_body(da_ref, db_ref, x_ref, xs_ref, r_ref):
    deg = jnp.maximum(da_ref[...] + db_ref[...], 1.0)
    r = lax.rsqrt(deg)
    r_ref[...] = r
    xs_ref[...] = x_ref[...] * r


def _scale_call(da, db, x):
    return pl.pallas_call(
        _scale_body,
        grid=(N // BR,),
        in_specs=[
            pl.BlockSpec((BR, 1), lambda i: (i, 0)),
            pl.BlockSpec((BR, 1), lambda i: (i, 0)),
            pl.BlockSpec((BR, D), lambda i: (i, 0)),
        ],
        out_specs=[
            pl.BlockSpec((BR, D), lambda i: (i, 0)),
            pl.BlockSpec((BR, 1), lambda i: (i, 0)),
        ],
        out_shape=[
            jax.ShapeDtypeStruct((N, D), jnp.float32),
            jax.ShapeDtypeStruct((N, 1), jnp.float32),
        ],
    )(da, db, x)


@functools.partial(
    pl.kernel,
    out_type=jax.ShapeDtypeStruct((NC, NPAD, D), jnp.float32),
    mesh=_mesh,
    scratch_types=[
        pltpu.VMEM((EPT_P,), jnp.int32),
        pltpu.VMEM((NBUF, K), jnp.int32),
        pltpu.VMEM((NBUF, K, D), jnp.float32),
        pltpu.VMEM((32, D), jnp.float32),
        pltpu.VMEM_SHARED((NPAD, D), jnp.float32),
        pltpu.SemaphoreType.DMA((NBUF,)),
        pltpu.SemaphoreType.DMA((NBUF,)),
        pltpu.SemaphoreType.DMA((NBUF,)),
    ],
)
def _agg_kernel(xs_hbm, src_hbm, dst2d_hbm, out_hbm,
                sidx_v, didx_v, rows_v, zero_v, agg_sh, gsem, ssem, isem):
    c = lax.axis_index("c")
    s = lax.axis_index("s")
    wid = s * NC + c
    r0 = s * RPT
    row0 = wid * NCHUNK

    for jr in range(32):
        for jc in range(D // 16):
            zero_v[jr, pl.ds(16 * jc, 16)] = jnp.zeros((16,), jnp.float32)
    for j in range(RPT // 32):
        pltpu.async_copy(
            zero_v, agg_sh.at[pl.ds(pl.multiple_of(r0 + 32 * j, 32), 32)],
            gsem.at[0])
    pltpu.sync_copy(src_hbm.at[pl.ds(wid * EPT_P, EPT_P)], sidx_v)
    for j in range(RPT // 32):
        pltpu.make_async_copy(
            zero_v, agg_sh.at[pl.ds(pl.multiple_of(r0 + 32 * j, 32), 32)],
            gsem.at[0]).wait()
    plsc.subcore_barrier()

    def fetch(i, b):
        off = pl.multiple_of(i * K, 8)
        pltpu.async_copy(xs_hbm.at[sidx_v.at[pl.ds(off, K)]], rows_v.at[b],
                         gsem.at[b])
        pltpu.async_copy(dst2d_hbm.at[row0 + i], didx_v.at[b], isem.at[b])

    def fetch_wait(i, b):
        off = pl.multiple_of(i * K, 8)
        pltpu.make_async_copy(xs_hbm.at[sidx_v.at[pl.ds(off, K)]],
                              rows_v.at[b], gsem.at[b]).wait()
        pltpu.make_async_copy(dst2d_hbm.at[row0 + i], didx_v.at[b],
                              isem.at[b]).wait()

    def scatter(b):
        pltpu.async_copy(rows_v.at[b], agg_sh.at[didx_v.at[b]], ssem.at[b],
                         add=True)

    def scatter_wait(b):
        pltpu.make_async_copy(rows_v.at[b], agg_sh.at[didx_v.at[b]],
                              ssem.at[b]).wait()

    for b in range(HALF):
        fetch(b, b)

    def body(g, carry):
        i0 = g * NBUF
        for b in range(HALF):
            fetch_wait(i0 + b, b)
            scatter(b)
        for b in range(HALF):
            fetch(i0 + HALF + b, HALF + b)
        for b in range(HALF):
            scatter_wait(b)
        for b in range(HALF):
            fetch_wait(i0 + HALF + b, HALF + b)
            scatter(HALF + b)
        @pl.when(g < NROUND - 1)
        def _():
            for b in range(HALF):
                fetch(i0 + NBUF + b, b)
        for b in range(HALF):
            scatter_wait(HALF + b)
        return carry

    lax.fori_loop(0, NROUND, body, 0)
    plsc.subcore_barrier()

    pltpu.sync_copy(agg_sh.at[pl.ds(r0, RPT)], out_hbm.at[c, pl.ds(r0, RPT)])


def _ffn_body(p0_ref, p1_ref, r_ref, w1_ref, b1_ref, w2_ref, b2_ref, out_ref):
    a = (p0_ref[0] + p1_ref[0]) * r_ref[...]
    h = jnp.maximum(jnp.dot(a, w1_ref[...],
                            preferred_element_type=jnp.float32) + b1_ref[...],
                    0.0)
    out_ref[...] = jnp.dot(h, w2_ref[...],
                           preferred_element_type=jnp.float32) + b2_ref[...]


def _ffn_call(agg_p, r, w1, b1, w2, b2):
    full = lambda i: (0, 0)
    return pl.pallas_call(
        _ffn_body,
        grid=(N // BR,),
        in_specs=[
            pl.BlockSpec((1, BR, D), lambda i: (0, i, 0)),
            pl.BlockSpec((1, BR, D), lambda i: (1, i, 0)),
            pl.BlockSpec((BR, 1), lambda i: (i, 0)),
            pl.BlockSpec((D, D), full),
            pl.BlockSpec((1, D), full),
            pl.BlockSpec((D, D), full),
            pl.BlockSpec((1, D), full),
        ],
        out_specs=pl.BlockSpec((BR, D), lambda i: (i, 0)),
        out_shape=jax.ShapeDtypeStruct((N, D), jnp.float32),
    )(agg_p, agg_p, r, w1, b1, w2, b2)


def kernel(x, edge_index, W1, b1, W2, b2):
    src = edge_index[0]
    dst = edge_index[1]

    pad_ids = jnp.arange(PAD_E, dtype=jnp.int32)
    pad_src = pad_ids % N
    pad_dst = N + pad_ids % (NPAD - N)
    src_agg = jnp.concatenate([src, pad_src])
    src_deg2d = jnp.concatenate([src, pad_dst]).reshape(NW * NCHUNK, K)
    dst_agg2d = jnp.concatenate([dst, pad_dst]).reshape(NW * NCHUNK, K)

    deg_p = _degree_kernel(src_deg2d)
    da = deg_p[0].reshape(NPAD, 1)
    db = deg_p[1].reshape(NPAD, 1)
    xs, r = _scale_call(da, db, x)

    agg_p = _agg_kernel(xs, src_agg, dst_agg2d)

    return _ffn_call(agg_p, r,
                     W1, b1.reshape(1, D), W2, b2.reshape(1, D))
